# baseline (speedup 1.0000x reference)
import functools

import jax
import jax.numpy as jnp
from jax import lax
from jax.experimental import pallas as pl
from jax.experimental.pallas import tpu as pltpu
from jax.experimental.pallas import tpu_sc as plsc

F = 26
D = 32
B = 16384
BF = B * F

_info = plsc.get_sparse_core_info()
NC, NS = _info.num_cores, _info.num_subcores
NW = NC * NS
NR = BF // NW
SCH = 416
NSCH = NR // SCH
DP = 128

NBUF = 4
GA = 2


def _gather_body(idx_hbm, table_hbm, out_hbm, idx_v, rows_v, *sems):
    gsems, wsems = sems[:NBUF], sems[NBUF:]
    wid = lax.axis_index("s") * NC + lax.axis_index("c")
    base = wid * NR
    pltpu.sync_copy(idx_hbm.at[pl.ds(base, NR)], idx_v)

    def fire_gather(s):
        b = s % NBUF
        return pltpu.async_copy(
            table_hbm.at[idx_v.at[pl.ds(s * SCH, SCH)]], rows_v.at[b], gsems[b]
        )

    def fire_write(s):
        b = s % NBUF
        return pltpu.async_copy(
            rows_v.at[b].at[:, pl.ds(0, D)],
            out_hbm.at[pl.ds(base + s * SCH, SCH)],
            wsems[b],
        )

    ghandles = [None] * NSCH
    whandles = [None] * NSCH
    for s in range(GA):
        ghandles[s] = fire_gather(s)
    for s in range(NSCH):
        ghandles[s].wait()
        whandles[s] = fire_write(s)
        t = s + GA
        if t < NSCH:
            if t >= NBUF:
                whandles[t - NBUF].wait()
            ghandles[t] = fire_gather(t)
    for s in range(NSCH - NBUF, NSCH):
        whandles[s].wait()


@jax.jit
def kernel(x, table, offsets):
    idx = (x + offsets[None, :]).reshape(BF)
    eye_pad = jnp.eye(D, DP, dtype=jnp.float32)
    t16 = jnp.dot(table, eye_pad).astype(jnp.bfloat16)
    mesh = plsc.VectorSubcoreMesh(core_axis_name="c", subcore_axis_name="s")
    run = pl.kernel(
        _gather_body,
        mesh=mesh,
        out_type=jax.ShapeDtypeStruct((BF, D), jnp.bfloat16),
        scratch_types=[
            pltpu.VMEM((NR,), jnp.int32),
            pltpu.VMEM((NBUF, SCH, DP), jnp.bfloat16),
        ]
        + [pltpu.SemaphoreType.DMA] * (2 * NBUF),
        compiler_params=pltpu.CompilerParams(use_tc_tiling_on_sc=False),
    )
    out = run(idx, t16)
    return out.astype(jnp.float32).reshape(B, F * D)


# speedup vs baseline: 4.0845x; 4.0845x over previous
import functools

import jax
import jax.numpy as jnp
from jax import lax
from jax.experimental import pallas as pl
from jax.experimental.pallas import tpu as pltpu
from jax.experimental.pallas import tpu_sc as plsc

F = 26
D = 32
B = 16384
BF = B * F

_info = plsc.get_sparse_core_info()
NC, NS = _info.num_cores, _info.num_subcores
NW = NC * NS
NR = BF // NW
SCH = 208
NSCH = NR // SCH
DP = 128

NBUF = 4
GA = 2


def _gather_body(idx_hbm, table_hbm, out_hbm, idx_v, rows_v, *sems):
    gsems, wsems = sems[:NBUF], sems[NBUF:]
    wid = lax.axis_index("s") * NC + lax.axis_index("c")
    base = wid * NR
    pltpu.sync_copy(idx_hbm.at[pl.ds(base, NR)], idx_v)

    def fire_gather(s):
        b = s % NBUF
        return pltpu.async_copy(
            table_hbm.at[idx_v.at[pl.ds(s * SCH, SCH)]], rows_v.at[b], gsems[b]
        )

    def fire_write(s):
        b = s % NBUF
        return pltpu.async_copy(
            rows_v.at[b].at[:, pl.ds(0, D)],
            out_hbm.at[pl.ds(base + s * SCH, SCH)],
            wsems[b],
        )

    ghandles = [None] * NSCH
    whandles = [None] * NSCH
    for s in range(GA):
        ghandles[s] = fire_gather(s)
    for s in range(NSCH):
        ghandles[s].wait()
        whandles[s] = fire_write(s)
        t = s + GA
        if t < NSCH:
            if t >= NBUF:
                whandles[t - NBUF].wait()
            ghandles[t] = fire_gather(t)
    for s in range(NSCH - NBUF, NSCH):
        whandles[s].wait()


@jax.jit
def kernel(x, table, offsets):
    idx = (x + offsets[None, :]).reshape(BF)
    eye_pad = jnp.eye(D, DP, dtype=jnp.float32)
    t128 = jnp.dot(table, eye_pad)
    mesh = plsc.VectorSubcoreMesh(core_axis_name="c", subcore_axis_name="s")
    run = pl.kernel(
        _gather_body,
        mesh=mesh,
        out_type=jax.ShapeDtypeStruct((BF, D), jnp.float32),
        scratch_types=[
            pltpu.VMEM((NR,), jnp.int32),
            pltpu.VMEM((NBUF, SCH, DP), jnp.float32),
        ]
        + [pltpu.SemaphoreType.DMA] * (2 * NBUF),
        compiler_params=pltpu.CompilerParams(use_tc_tiling_on_sc=False),
    )
    out = run(idx, t128)
    return out.reshape(B, F * D)
